# baseline (speedup 1.0000x reference)
import functools

import jax
import jax.numpy as jnp
from jax import lax
from jax.experimental import pallas as pl
from jax.experimental.pallas import tpu as pltpu
from jax.experimental.pallas import tpu_sc as plsc

_L = 16


@functools.lru_cache(maxsize=None)
def _build(B, H, W, D):
    C = 2 * D

    mesh = plsc.VectorSubcoreMesh(core_axis_name="c", subcore_axis_name="s")

    @functools.partial(
        pl.kernel,
        mesh=mesh,
        out_type=jax.ShapeDtypeStruct((B, H, W, C), jnp.float32),
        scratch_types=[
            pltpu.VMEM((W, C), jnp.float32),
            pltpu.VMEM((1, D), jnp.float32),
            pltpu.VMEM_SHARED((16, W, C), jnp.float32),
            pltpu.SemaphoreType.DMA,
        ],
    )
    def pos_embed(row_hbm, col_hbm, out_hbm, plane, rowv, shared, sem):
        s = lax.axis_index("s")
        i = lax.axis_index("c") * 16 + s

        pltpu.sync_copy(col_hbm.at[pl.ds(0, W)], plane.at[:, pl.ds(0, D)])
        pltpu.sync_copy(row_hbm.at[pl.ds(i, 1)], rowv)

        segs = [rowv[0, pl.ds(k * _L, _L)] for k in range(D // _L)]
        for j in range(W):
            for k, v in enumerate(segs):
                plane[j, pl.ds(D + k * _L, _L)] = v

        pltpu.sync_copy(plane, shared.at[s])
        plsc.subcore_barrier()

        copies = [
            pltpu.async_copy(plane, out_hbm.at[b, i], sem)
            for b in range(B // 2)
        ] + [
            pltpu.async_copy(shared.at[s], out_hbm.at[b, i], sem)
            for b in range(B // 2, B)
        ]
        for cp in copies:
            cp.wait()

    return pos_embed


def kernel(x, row_embed, col_embed):
    B = x.shape[0]
    H, W = x.shape[-2], x.shape[-1]
    D = row_embed.shape[1]
    out = _build(B, H, W, D)(row_embed, col_embed)
    return jnp.transpose(out, (0, 3, 1, 2))


# speedup vs baseline: 1.0553x; 1.0553x over previous
import functools

import jax
import jax.numpy as jnp
from jax import lax
from jax.experimental import pallas as pl
from jax.experimental.pallas import tpu as pltpu
from jax.experimental.pallas import tpu_sc as plsc

_L = 16


@functools.lru_cache(maxsize=None)
def _build(B, H, W, D):
    C = 2 * D

    mesh = plsc.VectorSubcoreMesh(core_axis_name="c", subcore_axis_name="s")

    @functools.partial(
        pl.kernel,
        mesh=mesh,
        out_type=jax.ShapeDtypeStruct((B, H, W, C), jnp.float32),
        scratch_types=[
            pltpu.VMEM((W, C), jnp.float32),
            pltpu.VMEM((1, D), jnp.float32),
            pltpu.SemaphoreType.DMA,
        ],
    )
    def pos_embed(row_hbm, col_hbm, out_hbm, plane, rowv, sem):
        i = lax.axis_index("c") * 16 + lax.axis_index("s")

        pltpu.sync_copy(col_hbm.at[pl.ds(0, W)], plane.at[:, pl.ds(0, D)])
        pltpu.sync_copy(row_hbm.at[pl.ds(i, 1)], rowv)

        segs = [rowv[0, pl.ds(k * _L, _L)] for k in range(D // _L)]
        for j in range(W):
            for k, v in enumerate(segs):
                plane[j, pl.ds(D + k * _L, _L)] = v

        copies = [
            pltpu.async_copy(plane, out_hbm.at[b, i], sem) for b in range(B)
        ]
        for cp in copies:
            cp.wait()

    return pos_embed


def kernel(x, row_embed, col_embed):
    B = x.shape[0]
    H, W = x.shape[-2], x.shape[-1]
    D = row_embed.shape[1]
    out = _build(B, H, W, D)(row_embed, col_embed)
    return jnp.transpose(out, (0, 3, 1, 2))
